# Initial kernel scaffold; baseline (speedup 1.0000x reference)
import functools

import jax
import jax.numpy as jnp
from jax import lax
from jax.experimental import pallas as pl
from jax.experimental.pallas import tpu as pltpu
from jax.experimental.pallas import tpu_sc as plsc

NC, NS, L = 2, 16, 16
NW = NC * NS


def _fm_sc(idx_flat, w, v, b, *, B, F, K):
    EPW = B // NW
    IPW = EPW * F
    E_CH = 64
    ROWS = E_CH * F
    NCHUNK = EPW // E_CH

    mesh = plsc.VectorSubcoreMesh(
        core_axis_name="c", subcore_axis_name="s", num_cores=NC, num_subcores=NS
    )

    @functools.partial(
        pl.kernel,
        out_type=jax.ShapeDtypeStruct((B,), jnp.float32),
        mesh=mesh,
        scratch_types=[
            pltpu.VMEM((IPW,), jnp.int32),
            pltpu.VMEM((ROWS, K), jnp.float32),
            pltpu.VMEM((ROWS, K), jnp.float32),
            pltpu.VMEM((ROWS, 1), jnp.float32),
            pltpu.VMEM((ROWS, 1), jnp.float32),
            pltpu.VMEM((EPW,), jnp.float32),
            pltpu.VMEM((EPW,), jnp.float32),
            pltpu.VMEM((EPW,), jnp.float32),
            pltpu.VMEM((8,), jnp.float32),
            pltpu.SemaphoreType.DMA,
            pltpu.SemaphoreType.DMA,
            pltpu.SemaphoreType.DMA,
            pltpu.SemaphoreType.DMA,
        ],
    )
    def fm_kernel(idx_hbm, w_hbm, v_hbm, b_hbm, out_hbm,
                  idxv, g0, g1, w0, w1, rbuf, lbuf, obuf, bbuf,
                  sv0, sv1, sw0, sw1):
        wid = lax.axis_index("s") * NC + lax.axis_index("c")
        base = wid * IPW

        pltpu.sync_copy(idx_hbm.at[pl.ds(base, IPW)], idxv)
        pltpu.sync_copy(b_hbm, bbuf.at[pl.ds(0, 1)])

        gbufs, wbufs, svs, sws = (g0, g1), (w0, w1), (sv0, sv1), (sw0, sw1)

        def start_gather(g):
            sel = g % 2
            idx_slice = idxv.at[pl.ds(g * ROWS, ROWS)]
            cv = pltpu.async_copy(v_hbm.at[idx_slice], gbufs[sel], svs[sel])
            cw = pltpu.async_copy(w_hbm.at[idx_slice], wbufs[sel], sws[sel])
            return cv, cw

        def compute_chunk(g):
            sel = g % 2
            gb, wb = gbufs[sel], wbufs[sel]

            @pl.loop(0, E_CH)
            def _(e):
                r0 = e * F
                s = gb[r0]
                ss = s * s
                lsum = wb[r0, 0]
                for f in range(1, F):
                    row = gb[r0 + f]
                    s += row
                    ss += row * row
                    lsum += wb[r0 + f, 0]
                eg = g * E_CH + e
                rbuf[eg] = jnp.sum(s * s - ss)
                lbuf[eg] = lsum

        pending = start_gather(0)
        for g in range(NCHUNK):
            cv, cw = pending
            cv.wait()
            cw.wait()
            if g + 1 < NCHUNK:
                pending = start_gather(g + 1)
            compute_chunk(g)

        bias = bbuf[0]

        @pl.loop(0, EPW, step=L)
        def _(i):
            logits = lbuf[pl.ds(i, L)] + bias + 0.5 * rbuf[pl.ds(i, L)]
            obuf[pl.ds(i, L)] = 1.0 / (1.0 + jnp.exp(-logits))

        pltpu.sync_copy(obuf, out_hbm.at[pl.ds(wid * EPW, EPW)])

    return fm_kernel(idx_flat, w, v, b)


def kernel(inputs, w, v, b):
    B, F = inputs.shape
    K = v.shape[1]
    idx_flat = inputs.reshape(B * F)
    return _fm_sc(idx_flat, w, v, b, B=B, F=F, K=K)


# speedup vs baseline: 1.3804x; 1.3804x over previous
import dataclasses
import functools

import jax
import jax.numpy as jnp
import numpy as np
from jax import lax
from jax.experimental import pallas as pl
from jax.experimental.pallas import tpu as pltpu
from jax.experimental.pallas import tpu_sc as plsc

NC, NS, L = 2, 16, 16
NW = NC * NS


def _fm_sc(idx_flat, w1d, v, b16, *, B, F, K):
    EPW = B // NW
    IPW = EPW * F
    E_CH = 64
    ROWS = E_CH * F
    NCHUNK = EPW // E_CH
    NGRP = E_CH // L

    mesh = plsc.VectorSubcoreMesh(
        core_axis_name="c", subcore_axis_name="s", num_cores=NC, num_subcores=NS
    )

    cp = pltpu.CompilerParams(use_tc_tiling_on_sc=False)
    if "needs_layout_passes" in pltpu.CompilerParams.__dataclass_fields__:
        cp = dataclasses.replace(cp, needs_layout_passes=False)

    @functools.partial(
        pl.kernel,
        out_type=jax.ShapeDtypeStruct((B,), jnp.float32),
        mesh=mesh,
        compiler_params=cp,
        scratch_types=[
            pltpu.VMEM((IPW,), jnp.int32),
            pltpu.VMEM((ROWS, K), jnp.float32),
            pltpu.VMEM((ROWS, K), jnp.float32),
            pltpu.VMEM((ROWS + L,), jnp.float32),
            pltpu.VMEM((ROWS + L,), jnp.float32),
            pltpu.VMEM((EPW,), jnp.float32),
            pltpu.VMEM((EPW,), jnp.float32),
            pltpu.VMEM((L,), jnp.float32),
            pltpu.SemaphoreType.DMA,
            pltpu.SemaphoreType.DMA,
            pltpu.SemaphoreType.DMA,
            pltpu.SemaphoreType.DMA,
        ],
    )
    def fm_kernel(idx_hbm, w_hbm, v_hbm, b_hbm, out_hbm,
                  idxv, g0, g1, w0, w1, tbuf, obuf, bbuf,
                  sv0, sv1, sw0, sw1):
        wid = lax.axis_index("s") * NC + lax.axis_index("c")

        pltpu.sync_copy(idx_hbm.at[pl.ds(wid * IPW, IPW)], idxv)
        pltpu.sync_copy(b_hbm, bbuf)

        gbufs, wbufs, svs, sws = (g0, g1), (w0, w1), (sv0, sv1), (sw0, sw1)

        def start_gather(g):
            sel = g % 2
            idx_slice = idxv.at[pl.ds(g * ROWS, ROWS)]
            cv = pltpu.async_copy(v_hbm.at[idx_slice], gbufs[sel], svs[sel])
            cw = pltpu.async_copy(
                w_hbm.at[idx_slice], wbufs[sel].at[pl.ds(0, ROWS)], sws[sel])
            return cv, cw

        lane = lax.iota(jnp.int32, L)

        def compute_chunk(g):
            sel = g % 2
            gb, wb = gbufs[sel], wbufs[sel]

            @pl.loop(0, NGRP)
            def _(grp):
                gbase = grp * (L * F)
                acc = None
                for j in range(L):
                    r0 = gbase + j * F
                    s = gb[r0]
                    ss = s * s
                    for f in range(1, F):
                        row = gb[r0 + f]
                        s += row
                        ss += row * row
                    o = (j * F) % 8
                    a0 = gbase + (j * F - o)
                    wa = wb[pl.ds(a0, L)]
                    wc = wb[pl.ds(a0 + L, L)]
                    mask_a = (lane >= o).astype(jnp.float32)
                    mask_b = (lane < (o + F - L)).astype(jnp.float32)
                    c = 0.5 * (s * s - ss) + wa * mask_a + wc * mask_b
                    onehot_j = (lane == j).astype(jnp.float32)
                    term = onehot_j * jnp.sum(c)
                    acc = term if acc is None else acc + term
                tbuf[pl.ds(g * E_CH + grp * L, L)] = acc

        pending = start_gather(0)
        for g in range(NCHUNK):
            cv, cw = pending
            cv.wait()
            cw.wait()
            if g + 1 < NCHUNK:
                pending = start_gather(g + 1)
            compute_chunk(g)

        bias = bbuf[pl.ds(0, L)]

        @pl.loop(0, EPW, step=L)
        def _(i):
            logits = tbuf[pl.ds(i, L)] + bias
            obuf[pl.ds(i, L)] = 1.0 / (1.0 + jnp.exp(-logits))

        pltpu.sync_copy(obuf, out_hbm.at[pl.ds(wid * EPW, EPW)])

    return fm_kernel(idx_flat, w1d, v, b16)


def kernel(inputs, w, v, b):
    B, F = inputs.shape
    K = v.shape[1]
    idx_flat = inputs.reshape(B * F)
    w1d = w.reshape(-1)
    b16 = jnp.broadcast_to(b, (L,))
    return _fm_sc(idx_flat, w1d, v, b16, B=B, F=F, K=K)
